# Initial kernel scaffold; baseline (speedup 1.0000x reference)
import jax
import jax.numpy as jnp
from jax.experimental import pallas as pl


def kernel(x, table):
    raise NotImplementedError("write your pallas kernel here")


# speedup vs baseline: 1.4667x; 1.4667x over previous
import functools

import jax
import jax.numpy as jnp
from jax import lax
from jax.experimental import pallas as pl
from jax.experimental.pallas import tpu as pltpu
from jax.experimental.pallas import tpu_sc as plsc

_D = 128
_C = 128
_L = 16


@functools.cache
def _build_sc_kernel(n_triples: int):
    info = plsc.get_sparse_core_info()
    nc, ns = info.num_cores, info.num_subcores
    nw = nc * ns
    per_w = n_triples // nw
    assert per_w * nw == n_triples and per_w % _C == 0
    n_chunks = per_w // _C
    mesh = plsc.VectorSubcoreMesh(core_axis_name="c", subcore_axis_name="s")

    @functools.partial(
        pl.kernel,
        mesh=mesh,
        out_type=jax.ShapeDtypeStruct((n_triples,), jnp.float32),
        compiler_params=pltpu.CompilerParams(needs_layout_passes=False),
        scratch_types=[
            pltpu.VMEM((_C,), jnp.int32),
            pltpu.VMEM((_C,), jnp.int32),
            pltpu.VMEM((_C,), jnp.int32),
            pltpu.VMEM((_C, _D), jnp.float32),
            pltpu.VMEM((_C, _D), jnp.float32),
            pltpu.VMEM((_C, _D), jnp.float32),
            pltpu.VMEM((2, _D), jnp.float32),
            pltpu.VMEM((_D,), jnp.float32),
            pltpu.VMEM((_C,), jnp.float32),
            pltpu.SemaphoreType.DMA,
        ],
    )
    def sc_entail(cl_hbm, cr_hbm, d_hbm, table_hbm, out_hbm,
                  cl_i, cr_i, d_i, cl_r, cr_r, d_r, bt_v, tbh_v, sc_v, sem):
        wid = lax.axis_index("s") * nc + lax.axis_index("c")
        base = wid * per_w

        pltpu.sync_copy(table_hbm.at[pl.ds(0, 2)], bt_v)
        for g in range(_D // _L):
            sl = pl.ds(g * _L, _L)
            tbh_v[sl] = 0.5 * (bt_v[1, sl] - bt_v[0, sl])

        def chunk_body(ch, carry):
            start = base + ch * _C
            pltpu.sync_copy(cl_hbm.at[pl.ds(start, _C)], cl_i)
            pltpu.sync_copy(cr_hbm.at[pl.ds(start, _C)], cr_i)
            pltpu.sync_copy(d_hbm.at[pl.ds(start, _C)], d_i)
            g1 = pltpu.async_copy(table_hbm.at[cl_i], cl_r, sem)
            g2 = pltpu.async_copy(table_hbm.at[cr_i], cr_r, sem)
            g3 = pltpu.async_copy(table_hbm.at[d_i], d_r, sem)
            g1.wait()
            g2.wait()
            g3.wait()

            for g16 in range(_C // _L):
                rows = (jnp.full((_L,), g16 * _L, jnp.int32)
                        + lax.iota(jnp.int32, _L))

                def col_body(c, carry2):
                    accd, acct = carry2
                    cols = jnp.full((_L,), c, jnp.int32)
                    a = plsc.load_gather(cl_r, [rows, cols])
                    b = plsc.load_gather(cr_r, [rows, cols])
                    dd = plsc.load_gather(d_r, [rows, cols])
                    tb = plsc.load_gather(tbh_v, [cols])
                    s = a + b
                    diff = 0.5 * s - dd
                    accd = accd + diff * diff
                    acct = acct + s * tb
                    return accd, acct

                accd, acct = lax.fori_loop(
                    0, _D, col_body,
                    (jnp.zeros((_L,), jnp.float32),
                     jnp.zeros((_L,), jnp.float32)))

                x = accd + 1e-12
                i = plsc.bitcast(x, jnp.int32)
                i = jnp.full((_L,), 0x5F3759DF, jnp.int32) - jnp.right_shift(i, 1)
                r = plsc.bitcast(i, jnp.float32)
                for _ in range(3):
                    r = r * (1.5 - 0.5 * x * r * r)
                sc_v[pl.ds(g16 * _L, _L)] = acct - x * r

            pltpu.sync_copy(sc_v, out_hbm.at[pl.ds(start, _C)])
            return carry

        lax.fori_loop(0, n_chunks, chunk_body, 0)

    return sc_entail


def kernel(x, table):
    bs, num_axioms, ents = x.shape
    assert ents == 3
    xt = x.reshape(-1, 3).astype(jnp.int32).T
    cl, cr, d = xt[0], xt[1], xt[2]
    scores = _build_sc_kernel(bs * num_axioms)(cl, cr, d, table)
    return scores.reshape(bs, num_axioms)


# speedup vs baseline: 1.8211x; 1.2416x over previous
import functools

import jax
import jax.numpy as jnp
from jax import lax
from jax.experimental import pallas as pl
from jax.experimental.pallas import tpu as pltpu
from jax.experimental.pallas import tpu_sc as plsc

_D = 128
_C = 128
_L = 16


@functools.cache
def _build_sc_kernel(n_triples: int):
    info = plsc.get_sparse_core_info()
    nc, ns = info.num_cores, info.num_subcores
    nw = nc * ns
    per_w = n_triples // nw
    assert per_w * nw == n_triples and per_w % (2 * _C) == 0
    n_half = per_w // (2 * _C)
    mesh = plsc.VectorSubcoreMesh(core_axis_name="c", subcore_axis_name="s")

    row_buf = pltpu.VMEM((_C, _D), jnp.float32)

    @functools.partial(
        pl.kernel,
        mesh=mesh,
        out_type=jax.ShapeDtypeStruct((n_triples,), jnp.float32),
        compiler_params=pltpu.CompilerParams(needs_layout_passes=False),
        scratch_types=[
            pltpu.VMEM((per_w,), jnp.int32),
            pltpu.VMEM((per_w,), jnp.int32),
            pltpu.VMEM((per_w,), jnp.int32),
            [row_buf, row_buf, row_buf],
            [row_buf, row_buf, row_buf],
            pltpu.VMEM((2, _D), jnp.float32),
            pltpu.VMEM((_D,), jnp.float32),
            pltpu.VMEM((_C,), jnp.float32),
            pltpu.SemaphoreType.DMA,
            pltpu.SemaphoreType.DMA,
        ],
    )
    def sc_entail(cl_hbm, cr_hbm, d_hbm, table_hbm, out_hbm,
                  cl_ia, cr_ia, d_ia, bufs0, bufs1, bt_v, tbh_v, sc_v,
                  sem0, sem1):
        wid = lax.axis_index("s") * nc + lax.axis_index("c")
        base = wid * per_w

        pltpu.sync_copy(table_hbm.at[pl.ds(0, 2)], bt_v)
        for g in range(_D // _L):
            sl = pl.ds(g * _L, _L)
            tbh_v[sl] = 0.5 * (bt_v[1, sl] - bt_v[0, sl])

        pltpu.sync_copy(cl_hbm.at[pl.ds(base, per_w)], cl_ia)
        pltpu.sync_copy(cr_hbm.at[pl.ds(base, per_w)], cr_ia)
        pltpu.sync_copy(d_hbm.at[pl.ds(base, per_w)], d_ia)

        idx_refs = (cl_ia, cr_ia, d_ia)

        def fire(bufs, sem, ch):
            s = pl.ds(ch * _C, _C)
            for ia, buf in zip(idx_refs, bufs):
                pltpu.async_copy(table_hbm.at[ia.at[s]], buf, sem)

        def drain(bufs, sem, ch):
            s = pl.ds(ch * _C, _C)
            for ia, buf in zip(idx_refs, bufs):
                pltpu.make_async_copy(table_hbm.at[ia.at[s]], buf, sem).wait()

        def compute(bufs, ch):
            cl_r, cr_r, d_r = bufs
            for g16 in range(_C // _L):
                rows = (jnp.full((_L,), g16 * _L, jnp.int32)
                        + lax.iota(jnp.int32, _L))

                def col_body(c, carry2):
                    accd, acct = carry2
                    cols = jnp.full((_L,), c, jnp.int32)
                    a = plsc.load_gather(cl_r, [rows, cols])
                    b = plsc.load_gather(cr_r, [rows, cols])
                    dd = plsc.load_gather(d_r, [rows, cols])
                    tb = plsc.load_gather(tbh_v, [cols])
                    s = a + b
                    diff = 0.5 * s - dd
                    accd = accd + diff * diff
                    acct = acct + s * tb
                    return accd, acct

                accd, acct = lax.fori_loop(
                    0, _D, col_body,
                    (jnp.zeros((_L,), jnp.float32),
                     jnp.zeros((_L,), jnp.float32)),
                    unroll=8)

                x = accd + 1e-12
                i = plsc.bitcast(x, jnp.int32)
                i = jnp.full((_L,), 0x5F3759DF, jnp.int32) - jnp.right_shift(i, 1)
                r = plsc.bitcast(i, jnp.float32)
                for _ in range(3):
                    r = r * (1.5 - 0.5 * x * r * r)
                sc_v[pl.ds(g16 * _L, _L)] = acct - x * r

            pltpu.sync_copy(sc_v, out_hbm.at[pl.ds(base + ch * _C, _C)])

        fire(bufs0, sem0, 0)

        def pair_body(ch2, carry):
            c0 = 2 * ch2
            fire(bufs1, sem1, c0 + 1)
            drain(bufs0, sem0, c0)
            compute(bufs0, c0)

            @pl.when(ch2 + 1 < n_half)
            def _():
                fire(bufs0, sem0, c0 + 2)

            drain(bufs1, sem1, c0 + 1)
            compute(bufs1, c0 + 1)
            return carry

        lax.fori_loop(0, n_half, pair_body, 0)

    return sc_entail


def kernel(x, table):
    bs, num_axioms, ents = x.shape
    assert ents == 3
    xt = x.reshape(-1, 3).astype(jnp.int32).T
    cl, cr, d = xt[0], xt[1], xt[2]
    scores = _build_sc_kernel(bs * num_axioms)(cl, cr, d, table)
    return scores.reshape(bs, num_axioms)


# speedup vs baseline: 11.3751x; 6.2464x over previous
import functools

import jax
import jax.numpy as jnp
from jax import lax
from jax.experimental import pallas as pl
from jax.experimental.pallas import tpu as pltpu
from jax.experimental.pallas import tpu_sc as plsc

_D = 128
_C = 128
_L = 16


@functools.cache
def _build_sc_kernel(n_triples: int):
    info = plsc.get_sparse_core_info()
    nc, ns = info.num_cores, info.num_subcores
    nw = nc * ns
    per_w = n_triples // nw
    assert per_w * nw == n_triples and per_w % (2 * _C) == 0
    n_half = per_w // (2 * _C)
    mesh = plsc.VectorSubcoreMesh(core_axis_name="c", subcore_axis_name="s")

    row_buf = pltpu.VMEM((_C, _D), jnp.float32)

    @functools.partial(
        pl.kernel,
        mesh=mesh,
        out_type=jax.ShapeDtypeStruct((n_triples,), jnp.float32),
        compiler_params=pltpu.CompilerParams(needs_layout_passes=False),
        scratch_types=[
            pltpu.VMEM((per_w,), jnp.int32),
            pltpu.VMEM((per_w,), jnp.int32),
            pltpu.VMEM((per_w,), jnp.int32),
            [row_buf, row_buf, row_buf],
            [row_buf, row_buf, row_buf],
            pltpu.VMEM((2, _D), jnp.float32),
            pltpu.VMEM((_D,), jnp.float32),
            pltpu.VMEM((_C,), jnp.float32),
            pltpu.SemaphoreType.DMA,
            pltpu.SemaphoreType.DMA,
        ],
    )
    def sc_entail(cl_hbm, cr_hbm, d_hbm, table_hbm, out_hbm,
                  cl_ia, cr_ia, d_ia, bufs0, bufs1, bt_v, tbh_v, sc_v,
                  sem0, sem1):
        wid = lax.axis_index("s") * nc + lax.axis_index("c")
        base = wid * per_w

        pltpu.sync_copy(table_hbm.at[pl.ds(0, 2)], bt_v)
        for g in range(_D // _L):
            sl = pl.ds(g * _L, _L)
            tbh_v[sl] = 0.5 * (bt_v[1, sl] - bt_v[0, sl])

        pltpu.sync_copy(cl_hbm.at[pl.ds(base, per_w)], cl_ia)
        pltpu.sync_copy(cr_hbm.at[pl.ds(base, per_w)], cr_ia)
        pltpu.sync_copy(d_hbm.at[pl.ds(base, per_w)], d_ia)

        idx_refs = (cl_ia, cr_ia, d_ia)

        def fire(bufs, sem, ch):
            s = pl.ds(ch * _C, _C)
            for ia, buf in zip(idx_refs, bufs):
                pltpu.async_copy(table_hbm.at[ia.at[s]], buf, sem)

        def drain(bufs, sem, ch):
            s = pl.ds(ch * _C, _C)
            for ia, buf in zip(idx_refs, bufs):
                pltpu.make_async_copy(table_hbm.at[ia.at[s]], buf, sem).wait()

        def compute(bufs, ch):
            cl_r, cr_r, d_r = bufs
            lanes = lax.iota(jnp.int32, _L)
            for g16 in range(_C // _L):
                rows = jnp.full((_L,), g16 * _L, jnp.int32) + lanes

                def col_body(c, carry2):
                    accd, acct = carry2
                    cols = jnp.bitwise_and(
                        jnp.full((_L,), c, jnp.int32) + lanes, _D - 1)
                    a = plsc.load_gather(cl_r, [rows, cols])
                    b = plsc.load_gather(cr_r, [rows, cols])
                    dd = plsc.load_gather(d_r, [rows, cols])
                    tb = plsc.load_gather(tbh_v, [cols])
                    s = a + b
                    diff = 0.5 * s - dd
                    accd = accd + diff * diff
                    acct = acct + s * tb
                    return accd, acct

                accd, acct = lax.fori_loop(
                    0, _D, col_body,
                    (jnp.zeros((_L,), jnp.float32),
                     jnp.zeros((_L,), jnp.float32)),
                    unroll=8)

                x = accd + 1e-12
                i = plsc.bitcast(x, jnp.int32)
                i = jnp.full((_L,), 0x5F3759DF, jnp.int32) - jnp.right_shift(i, 1)
                r = plsc.bitcast(i, jnp.float32)
                for _ in range(3):
                    r = r * (1.5 - 0.5 * x * r * r)
                sc_v[pl.ds(g16 * _L, _L)] = acct - x * r

            pltpu.sync_copy(sc_v, out_hbm.at[pl.ds(base + ch * _C, _C)])

        fire(bufs0, sem0, 0)

        def pair_body(ch2, carry):
            c0 = 2 * ch2
            fire(bufs1, sem1, c0 + 1)
            drain(bufs0, sem0, c0)
            compute(bufs0, c0)

            @pl.when(ch2 + 1 < n_half)
            def _():
                fire(bufs0, sem0, c0 + 2)

            drain(bufs1, sem1, c0 + 1)
            compute(bufs1, c0 + 1)
            return carry

        lax.fori_loop(0, n_half, pair_body, 0)

    return sc_entail


def kernel(x, table):
    bs, num_axioms, ents = x.shape
    assert ents == 3
    xt = x.reshape(-1, 3).astype(jnp.int32).T
    cl, cr, d = xt[0], xt[1], xt[2]
    scores = _build_sc_kernel(bs * num_axioms)(cl, cr, d, table)
    return scores.reshape(bs, num_axioms)


# speedup vs baseline: 12.4406x; 1.0937x over previous
import functools

import jax
import jax.numpy as jnp
from jax import lax
from jax.experimental import pallas as pl
from jax.experimental.pallas import tpu as pltpu
from jax.experimental.pallas import tpu_sc as plsc

_D = 128
_C = 128
_L = 16


@functools.cache
def _build_sc_kernel(n_triples: int):
    info = plsc.get_sparse_core_info()
    nc, ns = info.num_cores, info.num_subcores
    nw = nc * ns
    per_w = n_triples // nw
    assert per_w * nw == n_triples and per_w % (2 * _C) == 0
    n_half = per_w // (2 * _C)
    mesh = plsc.VectorSubcoreMesh(core_axis_name="c", subcore_axis_name="s")

    row_buf = pltpu.VMEM((_C, _D), jnp.float32)

    @functools.partial(
        pl.kernel,
        mesh=mesh,
        out_type=jax.ShapeDtypeStruct((n_triples,), jnp.float32),
        compiler_params=pltpu.CompilerParams(needs_layout_passes=False),
        scratch_types=[
            pltpu.VMEM((per_w,), jnp.int32),
            pltpu.VMEM((per_w,), jnp.int32),
            pltpu.VMEM((per_w,), jnp.int32),
            [row_buf, row_buf, row_buf],
            [row_buf, row_buf, row_buf],
            pltpu.VMEM((2, _D), jnp.float32),
            pltpu.VMEM((_D,), jnp.float32),
            pltpu.VMEM((_C,), jnp.float32),
            pltpu.SemaphoreType.DMA,
            pltpu.SemaphoreType.DMA,
        ],
    )
    def sc_entail(cl_hbm, cr_hbm, d_hbm, table_hbm, out_hbm,
                  cl_ia, cr_ia, d_ia, bufs0, bufs1, bt_v, tbh_v, sc_v,
                  sem0, sem1):
        wid = lax.axis_index("s") * nc + lax.axis_index("c")
        base = wid * per_w

        pltpu.sync_copy(table_hbm.at[pl.ds(0, 2)], bt_v)
        for g in range(_D // _L):
            sl = pl.ds(g * _L, _L)
            tbh_v[sl] = 0.5 * (bt_v[1, sl] - bt_v[0, sl])

        pltpu.sync_copy(cl_hbm.at[pl.ds(base, per_w)], cl_ia)
        pltpu.sync_copy(cr_hbm.at[pl.ds(base, per_w)], cr_ia)
        pltpu.sync_copy(d_hbm.at[pl.ds(base, per_w)], d_ia)

        idx_refs = (cl_ia, cr_ia, d_ia)

        def fire(bufs, sem, ch):
            s = pl.ds(ch * _C, _C)
            for ia, buf in zip(idx_refs, bufs):
                pltpu.async_copy(table_hbm.at[ia.at[s]], buf, sem)

        def drain(bufs, sem, ch):
            s = pl.ds(ch * _C, _C)
            for ia, buf in zip(idx_refs, bufs):
                pltpu.make_async_copy(table_hbm.at[ia.at[s]], buf, sem).wait()

        def compute(bufs, ch):
            cl_r, cr_r, d_r = bufs
            lanes = lax.iota(jnp.int32, _L)
            n_g = _C // _L
            rows_list = [jnp.full((_L,), g * _L, jnp.int32) + lanes
                         for g in range(n_g)]

            def col_body(c, carry2):
                cols = jnp.bitwise_and(
                    jnp.full((_L,), c, jnp.int32) + lanes, _D - 1)
                tb = plsc.load_gather(tbh_v, [cols])
                new = []
                for g in range(n_g):
                    a = plsc.load_gather(cl_r, [rows_list[g], cols])
                    b = plsc.load_gather(cr_r, [rows_list[g], cols])
                    dd = plsc.load_gather(d_r, [rows_list[g], cols])
                    s = a + b
                    diff = 0.5 * s - dd
                    new.append(carry2[2 * g] + diff * diff)
                    new.append(carry2[2 * g + 1] + s * tb)
                return tuple(new)

            accs = lax.fori_loop(
                0, _D, col_body,
                tuple(jnp.zeros((_L,), jnp.float32) for _ in range(2 * n_g)),
                unroll=2)

            for g in range(n_g):
                accd, acct = accs[2 * g], accs[2 * g + 1]
                x = accd + 1e-12
                i = plsc.bitcast(x, jnp.int32)
                i = jnp.full((_L,), 0x5F3759DF, jnp.int32) - jnp.right_shift(i, 1)
                r = plsc.bitcast(i, jnp.float32)
                for _ in range(3):
                    r = r * (1.5 - 0.5 * x * r * r)
                sc_v[pl.ds(g * _L, _L)] = acct - x * r

            pltpu.sync_copy(sc_v, out_hbm.at[pl.ds(base + ch * _C, _C)])

        fire(bufs0, sem0, 0)

        def pair_body(ch2, carry):
            c0 = 2 * ch2
            fire(bufs1, sem1, c0 + 1)
            drain(bufs0, sem0, c0)
            compute(bufs0, c0)

            @pl.when(ch2 + 1 < n_half)
            def _():
                fire(bufs0, sem0, c0 + 2)

            drain(bufs1, sem1, c0 + 1)
            compute(bufs1, c0 + 1)
            return carry

        lax.fori_loop(0, n_half, pair_body, 0)

    return sc_entail


def kernel(x, table):
    bs, num_axioms, ents = x.shape
    assert ents == 3
    xt = x.reshape(-1, 3).astype(jnp.int32).T
    cl, cr, d = xt[0], xt[1], xt[2]
    scores = _build_sc_kernel(bs * num_axioms)(cl, cr, d, table)
    return scores.reshape(bs, num_axioms)


# speedup vs baseline: 12.6717x; 1.0186x over previous
import functools

import jax
import jax.numpy as jnp
from jax import lax
from jax.experimental import pallas as pl
from jax.experimental.pallas import tpu as pltpu
from jax.experimental.pallas import tpu_sc as plsc

_D = 128
_C = 128
_L = 16


@functools.cache
def _build_sc_kernel(n_triples: int):
    info = plsc.get_sparse_core_info()
    nc, ns = info.num_cores, info.num_subcores
    nw = nc * ns
    per_w = n_triples // nw
    assert per_w * nw == n_triples and per_w % (2 * _C) == 0
    n_half = per_w // (2 * _C)
    mesh = plsc.VectorSubcoreMesh(core_axis_name="c", subcore_axis_name="s")

    row_buf = pltpu.VMEM((_C, _D), jnp.float32)

    @functools.partial(
        pl.kernel,
        mesh=mesh,
        out_type=jax.ShapeDtypeStruct((n_triples,), jnp.float32),
        compiler_params=pltpu.CompilerParams(needs_layout_passes=False),
        scratch_types=[
            pltpu.VMEM((per_w,), jnp.int32),
            pltpu.VMEM((per_w,), jnp.int32),
            pltpu.VMEM((per_w,), jnp.int32),
            [row_buf, row_buf, row_buf],
            [row_buf, row_buf, row_buf],
            pltpu.VMEM((2, _D), jnp.float32),
            pltpu.VMEM((_D,), jnp.float32),
            pltpu.VMEM((_C,), jnp.float32),
            pltpu.SemaphoreType.DMA,
            pltpu.SemaphoreType.DMA,
        ],
    )
    def sc_entail(cl_hbm, cr_hbm, d_hbm, table_hbm, out_hbm,
                  cl_ia, cr_ia, d_ia, bufs0, bufs1, bt_v, tbh_v, sc_v,
                  sem0, sem1):
        wid = lax.axis_index("s") * nc + lax.axis_index("c")
        base = wid * per_w

        pltpu.sync_copy(table_hbm.at[pl.ds(0, 2)], bt_v)
        for g in range(_D // _L):
            sl = pl.ds(g * _L, _L)
            tbh_v[sl] = 0.5 * (bt_v[1, sl] - bt_v[0, sl])

        pltpu.sync_copy(cl_hbm.at[pl.ds(base, per_w)], cl_ia)
        pltpu.sync_copy(cr_hbm.at[pl.ds(base, per_w)], cr_ia)
        pltpu.sync_copy(d_hbm.at[pl.ds(base, per_w)], d_ia)

        idx_refs = (cl_ia, cr_ia, d_ia)

        def fire(bufs, sem, ch):
            s = pl.ds(ch * _C, _C)
            for ia, buf in zip(idx_refs, bufs):
                pltpu.async_copy(table_hbm.at[ia.at[s]], buf, sem)

        def drain(bufs, sem, ch):
            s = pl.ds(ch * _C, _C)
            for ia, buf in zip(idx_refs, bufs):
                pltpu.make_async_copy(table_hbm.at[ia.at[s]], buf, sem).wait()

        def compute(bufs, ch):
            cl_r, cr_r, d_r = bufs
            lanes = lax.iota(jnp.int32, _L)
            n_g = _C // _L
            rows_list = [jnp.full((_L,), g * _L, jnp.int32) + lanes
                         for g in range(n_g)]

            def col_body(c, carry2):
                cols = jnp.bitwise_and(
                    jnp.full((_L,), c, jnp.int32) + lanes, _D - 1)
                tb = plsc.load_gather(tbh_v, [cols])
                new = []
                for g in range(n_g):
                    a = plsc.load_gather(cl_r, [rows_list[g], cols])
                    b = plsc.load_gather(cr_r, [rows_list[g], cols])
                    dd = plsc.load_gather(d_r, [rows_list[g], cols])
                    s = a + b
                    diff = 0.5 * s - dd
                    new.append(carry2[2 * g] + diff * diff)
                    new.append(carry2[2 * g + 1] + s * tb)
                return tuple(new)

            accs = lax.fori_loop(
                0, _D, col_body,
                tuple(jnp.zeros((_L,), jnp.float32) for _ in range(2 * n_g)),
                unroll=4)

            for g in range(n_g):
                accd, acct = accs[2 * g], accs[2 * g + 1]
                x = accd + 1e-12
                i = plsc.bitcast(x, jnp.int32)
                i = jnp.full((_L,), 0x5F3759DF, jnp.int32) - jnp.right_shift(i, 1)
                r = plsc.bitcast(i, jnp.float32)
                for _ in range(3):
                    r = r * (1.5 - 0.5 * x * r * r)
                sc_v[pl.ds(g * _L, _L)] = acct - x * r

            pltpu.sync_copy(sc_v, out_hbm.at[pl.ds(base + ch * _C, _C)])

        fire(bufs0, sem0, 0)

        def pair_body(ch2, carry):
            c0 = 2 * ch2
            fire(bufs1, sem1, c0 + 1)
            drain(bufs0, sem0, c0)
            compute(bufs0, c0)

            @pl.when(ch2 + 1 < n_half)
            def _():
                fire(bufs0, sem0, c0 + 2)

            drain(bufs1, sem1, c0 + 1)
            compute(bufs1, c0 + 1)
            return carry

        lax.fori_loop(0, n_half, pair_body, 0)

    return sc_entail


def kernel(x, table):
    bs, num_axioms, ents = x.shape
    assert ents == 3
    xt = x.reshape(-1, 3).astype(jnp.int32).T
    cl, cr, d = xt[0], xt[1], xt[2]
    scores = _build_sc_kernel(bs * num_axioms)(cl, cr, d, table)
    return scores.reshape(bs, num_axioms)
